# baseline (speedup 1.0000x reference)
import functools

import jax
import jax.numpy as jnp
from jax import lax
from jax.experimental import pallas as pl
from jax.experimental.pallas import tpu as pltpu
from jax.experimental.pallas import tpu_sc as plsc

VOCAB = 1000000
HIDDEN = 64
B = 4096
L = 200

NC = 2
NS = 16
NW = NC * NS

N = B * L
CH = 128
N_BLK = N // CH
BLK_W = N_BLK // NW
NBT = B // CH

_mesh = plsc.VectorSubcoreMesh(
    core_axis_name="c", subcore_axis_name="s", num_cores=NC, num_subcores=NS
)


@functools.partial(
    pl.kernel,
    out_type=jax.ShapeDtypeStruct((L, HIDDEN, B), jnp.float32),
    mesh=_mesh,
    scratch_types=[
        pltpu.VMEM((BLK_W, CH), jnp.int32),
        pltpu.VMEM((BLK_W, CH), jnp.int32),
        pltpu.VMEM((CH, 2 * HIDDEN), jnp.float32),
        pltpu.VMEM((CH, 2 * HIDDEN), jnp.float32),
        pltpu.VMEM((HIDDEN, CH), jnp.float32),
        pltpu.VMEM((HIDDEN, CH), jnp.float32),
        pltpu.SemaphoreType.DMA,
        pltpu.SemaphoreType.DMA,
        pltpu.SemaphoreType.DMA,
        pltpu.SemaphoreType.DMA,
    ],
    compiler_params=pltpu.CompilerParams(
        use_tc_tiling_on_sc=True, needs_layout_passes=False
    ),
)
def _embed_gather(pidx_hbm, hoff_hbm, tbl_hbm, out_hbm,
                  pidx_v, hoff_v, buf0, buf1, st0, st1,
                  sg0, sg1, so0, so1):
    wid = lax.axis_index("s") * NC + lax.axis_index("c")
    base_blk = wid * BLK_W
    bufs = (buf0, buf1)
    stages = (st0, st1)
    sgs = (sg0, sg1)
    sos = (so0, so1)

    pltpu.sync_copy(pidx_hbm.at[pl.ds(base_blk, BLK_W)], pidx_v)
    pltpu.sync_copy(hoff_hbm.at[pl.ds(base_blk, BLK_W)], hoff_v)

    iotas = tuple(
        lax.iota(jnp.int32, 16) + jnp.int32(16 * j) for j in range(CH // 16)
    )

    def fire(t, b):
        pltpu.async_copy(tbl_hbm.at[pidx_v.at[t]], bufs[b], sgs[b])

    def drain_gather(b):
        pltpu.make_async_copy(
            tbl_hbm.at[pl.ds(0, CH)], bufs[b], sgs[b]
        ).wait()

    def select_transpose(t, b):
        buf = bufs[b]
        stage = stages[b]

        def col(h, carry):
            for j in range(CH // 16):
                rows = iotas[j]
                cols = hoff_v[t, pl.ds(16 * j, 16)] + h
                v = plsc.load_gather(buf, [rows, cols])
                stage[h, pl.ds(16 * j, 16)] = v
            return carry

        lax.fori_loop(0, HIDDEN, col, 0)

    def start_out(t, b):
        g = base_blk + t
        l = g // NBT
        bt = g % NBT
        pltpu.async_copy(
            stages[b], out_hbm.at[l, :, pl.ds(bt * CH, CH)], sos[b]
        )

    def wait_out(b):
        pltpu.make_async_copy(
            stages[b], out_hbm.at[0, :, pl.ds(0, CH)], sos[b]
        ).wait()

    fire(0, 0)
    fire(1, 1)

    def step(k, carry):
        t = 2 * k
        drain_gather(0)
        select_transpose(t, 0)
        lax.cond(t + 2 < BLK_W, lambda: fire(t + 2, 0), lambda: None)
        lax.cond(t >= 2, lambda: wait_out(0), lambda: None)
        start_out(t, 0)
        drain_gather(1)
        select_transpose(t + 1, 1)
        lax.cond(t + 3 < BLK_W, lambda: fire(t + 3, 1), lambda: None)
        lax.cond(t + 1 >= 2, lambda: wait_out(1), lambda: None)
        start_out(t + 1, 1)
        return carry

    lax.fori_loop(0, BLK_W // 2, step, 0)
    wait_out(0)
    wait_out(1)


def kernel(input_ids, table):
    ids_t = input_ids.T.reshape(N_BLK, CH)
    pidx = lax.shift_right_logical(ids_t, 1)
    hoff = (ids_t & 1) * HIDDEN
    tbl2 = table.reshape(VOCAB // 2, 2 * HIDDEN)
    raw = _embed_gather(pidx, hoff, tbl2)
    return raw.transpose(2, 0, 1)


# speedup vs baseline: 1.1923x; 1.1923x over previous
import functools

import jax
import jax.numpy as jnp
from jax import lax
from jax.experimental import pallas as pl
from jax.experimental.pallas import tpu as pltpu
from jax.experimental.pallas import tpu_sc as plsc

VOCAB = 1000000
HIDDEN = 64
B = 4096
L = 200

NC = 2
NS = 16
NW = NC * NS

N = B * L
CH = 128
N_BLK = N // CH
BLK_W = N_BLK // NW
G = 5
NGRP = BLK_W // G
GROWS = G * CH

_mesh = plsc.VectorSubcoreMesh(
    core_axis_name="c", subcore_axis_name="s", num_cores=NC, num_subcores=NS
)


@functools.partial(
    pl.kernel,
    out_type=jax.ShapeDtypeStruct((N, HIDDEN), jnp.float32),
    mesh=_mesh,
    scratch_types=[
        pltpu.VMEM((BLK_W, CH), jnp.int32),
        pltpu.VMEM((GROWS, HIDDEN), jnp.float32),
        pltpu.VMEM((GROWS, HIDDEN), jnp.float32),
        pltpu.SemaphoreType.DMA,
        pltpu.SemaphoreType.DMA,
        pltpu.SemaphoreType.DMA,
        pltpu.SemaphoreType.DMA,
    ],
    compiler_params=pltpu.CompilerParams(use_tc_tiling_on_sc=False),
)
def _embed_lookup(ids_hbm, table_hbm, out_hbm, idx_v, buf0, buf1,
                  sg0, sg1, so0, so1):
    wid = lax.axis_index("s") * NC + lax.axis_index("c")
    base_blk = wid * BLK_W
    bufs = (buf0, buf1)
    sgs = (sg0, sg1)
    sos = (so0, so1)

    pltpu.sync_copy(ids_hbm.at[pl.ds(base_blk, BLK_W)], idx_v)

    def fire(t, b):
        for j in range(G):
            pltpu.async_copy(
                table_hbm.at[idx_v.at[t * G + j]],
                bufs[b].at[pl.ds(j * CH, CH)],
                sgs[b],
            )

    def drain_gathers(b):
        pltpu.make_async_copy(
            out_hbm.at[pl.ds(0, GROWS)], bufs[b], sgs[b]
        ).wait()

    def start_out(t, b):
        pltpu.async_copy(
            bufs[b], out_hbm.at[pl.ds((base_blk + t * G) * CH, GROWS)], sos[b]
        )

    def wait_out(b):
        pltpu.make_async_copy(
            bufs[b], out_hbm.at[pl.ds(0, GROWS)], sos[b]
        ).wait()

    fire(0, 0)
    fire(1, 1)
    drain_gathers(0)
    start_out(0, 0)

    def step(k, carry):
        t = 1 + 2 * k
        wait_out(0)
        fire(t + 1, 0)
        drain_gathers(1)
        start_out(t, 1)
        wait_out(1)
        fire(t + 2, 1)
        drain_gathers(0)
        start_out(t + 1, 0)
        return carry

    lax.fori_loop(0, (NGRP - 2) // 2, step, 0)

    drain_gathers(1)
    start_out(NGRP - 1, 1)
    wait_out(0)
    wait_out(1)


def kernel(input_ids, table):
    ids2 = (input_ids.astype(jnp.int32) * 2).reshape(N_BLK, CH)
    tbl_pad = jnp.pad(
        table.reshape(VOCAB, 1, HIDDEN), ((0, 0), (0, 1), (0, 0))
    ).reshape(2 * VOCAB, HIDDEN)
    out = _embed_lookup(ids2, tbl_pad)
    return out.reshape(B, L, HIDDEN)


# speedup vs baseline: 2.0517x; 1.7208x over previous
import functools

import jax
import jax.numpy as jnp
from jax import lax
from jax.experimental import pallas as pl
from jax.experimental.pallas import tpu as pltpu
from jax.experimental.pallas import tpu_sc as plsc

VOCAB = 1000000
HIDDEN = 64
B = 4096
L = 200

NC = 2
NS = 16
NW = NC * NS

N = B * L
CH = 128
N_BLK = N // CH
BLK_W = N_BLK // NW
G = 5
NGRP = BLK_W // G
GROWS = G * CH

_mesh = plsc.VectorSubcoreMesh(
    core_axis_name="c", subcore_axis_name="s", num_cores=NC, num_subcores=NS
)


@functools.partial(
    pl.kernel,
    out_type=jax.ShapeDtypeStruct((N, HIDDEN), jnp.float32),
    mesh=_mesh,
    scratch_types=[
        pltpu.VMEM((BLK_W, CH), jnp.int32),
        pltpu.VMEM((GROWS, HIDDEN), jnp.float32),
        pltpu.VMEM((GROWS, HIDDEN), jnp.float32),
        pltpu.SemaphoreType.DMA,
        pltpu.SemaphoreType.DMA,
        pltpu.SemaphoreType.DMA,
        pltpu.SemaphoreType.DMA,
    ],
    compiler_params=pltpu.CompilerParams(use_tc_tiling_on_sc=False),
)
def _embed_lookup(ids_hbm, table_hbm, out_hbm, idx_v, buf0, buf1,
                  sg0, sg1, so0, so1):
    wid = lax.axis_index("s") * NC + lax.axis_index("c")
    base_blk = wid * BLK_W
    bufs = (buf0, buf1)
    sgs = (sg0, sg1)
    sos = (so0, so1)

    pltpu.sync_copy(ids_hbm.at[pl.ds(base_blk, BLK_W)], idx_v)

    def fire(t, b):
        for j in range(G):
            pltpu.async_copy(
                table_hbm.at[idx_v.at[t * G + j]],
                bufs[b].at[pl.ds(j * CH, CH)],
                sgs[b],
            )

    def drain_gathers(b):
        pltpu.make_async_copy(
            out_hbm.at[pl.ds(0, GROWS)], bufs[b], sgs[b]
        ).wait()

    def start_out(t, b):
        pltpu.async_copy(
            bufs[b], out_hbm.at[pl.ds((base_blk + t * G) * CH, GROWS)], sos[b]
        )

    def wait_out(b):
        pltpu.make_async_copy(
            bufs[b], out_hbm.at[pl.ds(0, GROWS)], sos[b]
        ).wait()

    fire(0, 0)
    fire(1, 1)
    drain_gathers(0)
    start_out(0, 0)

    def step(k, carry):
        t = 1 + 2 * k
        wait_out(0)
        fire(t + 1, 0)
        drain_gathers(1)
        start_out(t, 1)
        wait_out(1)
        fire(t + 2, 1)
        drain_gathers(0)
        start_out(t + 1, 0)
        return carry

    lax.fori_loop(0, (NGRP - 2) // 2, step, 0)

    drain_gathers(1)
    start_out(NGRP - 1, 1)
    wait_out(0)
    wait_out(1)


def kernel(input_ids, table):
    ids = input_ids.astype(jnp.int32).reshape(N_BLK, CH)
    out = _embed_lookup(ids, table)
    return out.reshape(B, L, HIDDEN)
